# baseline (speedup 1.0000x reference)
import jax
import jax.numpy as jnp
from jax.experimental import pallas as pl

E = 8
TILE = 8192


def _router_body(x_ref, w_ref, wout_ref, iout_ref):
    logits = jnp.dot(x_ref[...], w_ref[...], preferred_element_type=jnp.float32)
    lt = logits.T

    e_iota = jax.lax.broadcasted_iota(jnp.int32, lt.shape, 0)
    m1 = jnp.max(lt, axis=0, keepdims=True)
    i1 = jnp.min(jnp.where(lt == m1, e_iota, E), axis=0, keepdims=True)
    masked = jnp.where(e_iota == i1, -jnp.inf, lt)
    m2 = jnp.max(masked, axis=0, keepdims=True)
    i2 = jnp.min(jnp.where(masked == m2, e_iota, E), axis=0, keepdims=True)

    z = jnp.sum(jnp.exp(lt - m1), axis=0, keepdims=True)
    p1 = 1.0 / z
    p2 = jnp.exp(m2 - m1) * p1
    w1 = jax.nn.sigmoid(p1 - p2)

    wout_ref[...] = jnp.concatenate([w1, 1.0 - w1], axis=0)
    iout_ref[...] = jnp.concatenate([i1, i2], axis=0)


def kernel(x, kernel_DE):
    B, T, D = x.shape
    N = B * T
    xf = x.reshape(N, D)
    wout, iout = pl.pallas_call(
        _router_body,
        grid=(N // TILE,),
        in_specs=[
            pl.BlockSpec((TILE, D), lambda i: (i, 0)),
            pl.BlockSpec((D, E), lambda i: (0, 0)),
        ],
        out_specs=[
            pl.BlockSpec((2, TILE), lambda i: (0, i)),
            pl.BlockSpec((2, TILE), lambda i: (0, i)),
        ],
        out_shape=[
            jax.ShapeDtypeStruct((2, N), jnp.float32),
            jax.ShapeDtypeStruct((2, N), jnp.int32),
        ],
    )(xf, kernel_DE)
    return wout.T.reshape(B, T, 2), iout.T.reshape(B, T, 2)


# speedup vs baseline: 1.0671x; 1.0671x over previous
import jax
import jax.numpy as jnp
from jax.experimental import pallas as pl

E = 8
TILE = 4096


def _router_body(x_ref, w_ref, wout_ref, iout_ref):
    logits = jnp.dot(x_ref[...], w_ref[...], preferred_element_type=jnp.float32)
    lt = logits.T

    e_iota = jax.lax.broadcasted_iota(jnp.int32, lt.shape, 0)
    m1 = jnp.max(lt, axis=0, keepdims=True)
    i1 = jnp.min(jnp.where(lt == m1, e_iota, E), axis=0, keepdims=True)
    masked = jnp.where(e_iota == i1, -jnp.inf, lt)
    m2 = jnp.max(masked, axis=0, keepdims=True)
    i2 = jnp.min(jnp.where(masked == m2, e_iota, E), axis=0, keepdims=True)

    z = jnp.sum(jnp.exp(lt - m1), axis=0, keepdims=True)
    p1 = 1.0 / z
    p2 = jnp.exp(m2 - m1) * p1
    w1 = jax.nn.sigmoid(p1 - p2)

    wout_ref[...] = jnp.concatenate([w1, 1.0 - w1], axis=0)
    iout_ref[...] = jnp.concatenate([i1, i2], axis=0)


def kernel(x, kernel_DE):
    B, T, D = x.shape
    N = B * T
    xf = x.reshape(N, D)
    wout, iout = pl.pallas_call(
        _router_body,
        grid=(N // TILE,),
        in_specs=[
            pl.BlockSpec((TILE, D), lambda i: (i, 0)),
            pl.BlockSpec((D, E), lambda i: (0, 0)),
        ],
        out_specs=[
            pl.BlockSpec((2, TILE), lambda i: (0, i)),
            pl.BlockSpec((2, TILE), lambda i: (0, i)),
        ],
        out_shape=[
            jax.ShapeDtypeStruct((2, N), jnp.float32),
            jax.ShapeDtypeStruct((2, N), jnp.int32),
        ],
    )(xf, kernel_DE)
    return wout.T.reshape(B, T, 2), iout.T.reshape(B, T, 2)
